# Initial kernel scaffold; baseline (speedup 1.0000x reference)
import jax
import jax.numpy as jnp
from jax.experimental import pallas as pl


def kernel(edge_index_SH, prescription, total_g, sub_g1, sub_g2, emb, W1, b1, W2, b2, Wm, bm, gamma, beta):
    raise NotImplementedError("write your pallas kernel here")


# speedup vs baseline: 6.2096x; 6.2096x over previous
import functools

import jax
import jax.numpy as jnp
from jax import lax
from jax.experimental import pallas as pl
from jax.experimental.pallas import tpu as pltpu
from jax.experimental.pallas import tpu_sc as plsc

_N = 1195
_E = 500000
_H = _E // 2
_F32 = jnp.float32
_DEF = lax.Precision.DEFAULT

_BH = [15680 * k for k in range(15)] + [234880, 250000]
_BOUNDS = _BH[:-1] + [_H + b for b in _BH[:-1]]
_WMAX = 15680
_ECH = 2048
_NCHW = (_WMAX + _ECH - 1) // _ECH
_EPAD = max(lo + _NCHW * _ECH for lo in _BOUNDS)
_MFLAT = _N * 64
_DW = 128
_ACC = _DW * 64

_CB = 1280
_CSL = _CB // 16
_HCH = 2048
_HNCH = (_E + _HCH - 1) // _HCH
_HEPAD = _HNCH * _HCH


def _sc_cnt_hist(dst_p, zeros_hbm):
    mesh = plsc.VectorSubcoreMesh(core_axis_name="c", subcore_axis_name="s",
                                  num_cores=2, num_subcores=16)

    @functools.partial(
        pl.kernel,
        mesh=mesh,
        out_type=jax.ShapeDtypeStruct((2 * _CB,), _F32),
        scratch_types=[
            pltpu.VMEM((_HCH,), jnp.int32),
            pltpu.VMEM((16, 128), jnp.int32),
            pltpu.VMEM((16, 128), _F32),
            pltpu.VMEM((_CSL,), _F32),
            pltpu.VMEM_SHARED((_CB,), _F32),
        ],
    )
    def build(dst_hbm, z_hbm, out_hbm, dst_v, idx_v, val_v, zbuf, acc):
        c = lax.axis_index("c")
        s = lax.axis_index("s")
        w = c * 16 + s
        pltpu.sync_copy(z_hbm.at[pl.ds(0, _CSL)], zbuf)
        pltpu.sync_copy(zbuf, acc.at[pl.ds(s * _CSL, _CSL)])
        plsc.subcore_barrier()

        lanes = lax.iota(jnp.int32, 16)

        def chunk_body(it, carry):
            ci = it * 32 + w

            @pl.when(ci < _HNCH)
            def _():
                base = ci * _HCH
                pltpu.sync_copy(dst_hbm.at[pl.ds(base, _HCH)], dst_v)
                for j in range(16):
                    for i in range(8):
                        off = j * 128 + i * 16
                        idx_v[j, pl.ds(i * 16, 16)] = dst_v[pl.ds(off, 16)]
                        eid = (base + off) + lanes
                        val_v[j, pl.ds(i * 16, 16)] = jnp.where(
                            eid < _E, jnp.float32(1.0), jnp.float32(0.0))
                for j in range(16):
                    pltpu.sync_copy(val_v.at[j], acc.at[idx_v.at[j]], add=True)
            return carry

        lax.fori_loop(0, (_HNCH + 31) // 32, chunk_body, 0)

        plsc.subcore_barrier()
        pltpu.sync_copy(acc.at[pl.ds(s * _CSL, _CSL)], zbuf)
        pltpu.sync_copy(zbuf, out_hbm.at[pl.ds(c * _CB + s * _CSL, _CSL)])

    return build(dst_p, zeros_hbm)


def _sc_segscan(ssrc_p, sdst_p, m_flat, zeros_hbm):
    mesh = plsc.VectorSubcoreMesh(core_axis_name="c", subcore_axis_name="s",
                                  num_cores=2, num_subcores=16)

    los = jnp.asarray(_BOUNDS, jnp.int32)
    his = jnp.asarray(_BOUNDS[1:] + [_E], jnp.int32)

    @functools.partial(
        pl.kernel,
        mesh=mesh,
        out_type=jax.ShapeDtypeStruct((32 * _ACC,), _F32),
        scratch_types=[
            pltpu.VMEM((_MFLAT,), _F32),
            pltpu.VMEM((_ECH + 16,), jnp.int32),
            pltpu.VMEM((_ECH + 16,), jnp.int32),
            pltpu.VMEM((_ACC,), _F32),
        ],
    )
    def scan(ssrc, sdst, mh, bl, bh, z_hbm, out, m_v, src_v, dst_v, acc_t):
        c = lax.axis_index("c")
        s = lax.axis_index("s")
        w = c * 16 + s

        pltpu.sync_copy(mh, m_v)
        pltpu.sync_copy(z_hbm, acc_t)

        pltpu.sync_copy(bl, src_v.at[pl.ds(0, 32)])
        pltpu.sync_copy(bh, src_v.at[pl.ds(32, 32)])
        def at(ref, i):
            b = pl.multiple_of((i // 2) * 2, 2)
            v = ref[pl.ds(b, 16)]
            return jnp.where(i == b, v[0], v[1])

        lo = pl.multiple_of(at(src_v, w), 16)
        hi = at(src_v, 32 + w)

        pltpu.sync_copy(sdst.at[pl.ds(lo, 16)], dst_v.at[pl.ds(0, 16)])
        d0 = (dst_v[pl.ds(0, 16)][0] // 8) * 8

        zero = jnp.zeros((16,), _F32)
        a0, a1, a2, a3 = zero, zero, zero, zero
        dprev = d0
        for ch in range(_NCHW):
            base = pl.multiple_of(lo + ch * _ECH, 16)
            pltpu.sync_copy(ssrc.at[pl.ds(base, _ECH)], src_v.at[pl.ds(0, _ECH)])
            pltpu.sync_copy(sdst.at[pl.ds(base, _ECH)], dst_v.at[pl.ds(0, _ECH)])
            nrem = jnp.clip(hi - base, 0, _ECH)

            def body(i, carry):
                a0, a1, a2, a3, dprev = carry
                si = at(src_v, i)
                di = at(dst_v, i)
                same = di == dprev

                @pl.when(jnp.logical_not(same))
                def _():
                    loc = pl.multiple_of(
                        jnp.minimum(dprev - d0, _DW - 1) * 64, 16)
                    acc_t[pl.ds(loc, 16)] = a0
                    acc_t[pl.ds(loc + 16, 16)] = a1
                    acc_t[pl.ds(loc + 32, 16)] = a2
                    acc_t[pl.ds(loc + 48, 16)] = a3

                ro = pl.multiple_of(si * 64, 16)
                a0 = jnp.where(same, a0, zero) + m_v[pl.ds(ro, 16)]
                a1 = jnp.where(same, a1, zero) + m_v[pl.ds(ro + 16, 16)]
                a2 = jnp.where(same, a2, zero) + m_v[pl.ds(ro + 32, 16)]
                a3 = jnp.where(same, a3, zero) + m_v[pl.ds(ro + 48, 16)]
                return a0, a1, a2, a3, di

            a0, a1, a2, a3, dprev = lax.fori_loop(
                0, nrem, body, (a0, a1, a2, a3, dprev))

        loc = pl.multiple_of(jnp.minimum(dprev - d0, _DW - 1) * 64, 16)
        acc_t[pl.ds(loc, 16)] = a0
        acc_t[pl.ds(loc + 16, 16)] = a1
        acc_t[pl.ds(loc + 32, 16)] = a2
        acc_t[pl.ds(loc + 48, 16)] = a3

        pltpu.sync_copy(acc_t, out.at[pl.ds(w * _ACC, _ACC)])

    return scan(ssrc_p, sdst_p, m_flat, los, his, zeros_hbm)


def _tc_lin(x, W, b):
    def body(x_ref, w_ref, b_ref, o_ref):
        o_ref[...] = lax.dot_general(
            x_ref[...], w_ref[...], (((1,), (1,)), ((), ())),
            precision=_DEF, preferred_element_type=_F32) + b_ref[...]
    return pl.pallas_call(
        body, out_shape=jax.ShapeDtypeStruct((x.shape[0], W.shape[0]), _F32),
    )(x, W, b)


def _merge_windows(p_ref, d0_ref, spad):
    spad[...] = jnp.zeros((_N + _DW, 64), _F32)
    for w in range(32):
        d0 = pl.multiple_of(d0_ref[w, 0], 8)
        blk = p_ref[w]
        spad[pl.ds(d0, _DW), :] = spad[pl.ds(d0, _DW), :] + blk
    return spad[0:_N, :]


def _tc_mid(p1, d0s, cnt_col, W2, b2):
    def body(p_ref, d0_ref, cnt_ref, w2_ref, b2_ref, o_ref, spad):
        s = _merge_windows(p_ref, d0_ref, spad)
        mean = s / jnp.maximum(cnt_ref[...], 1.0)
        x2 = jnp.tanh(mean)
        o_ref[...] = lax.dot_general(
            x2, w2_ref[...], (((1,), (1,)), ((), ())),
            precision=_DEF, preferred_element_type=_F32) + b2_ref[...]
    return pl.pallas_call(
        body,
        out_shape=jax.ShapeDtypeStruct((_N, 64), _F32),
        scratch_shapes=[pltpu.VMEM((_N + _DW, 64), _F32)],
    )(p1, d0s, cnt_col, W2, b2)


def _tc_tail(p2, d0s, cnt_col, presc, Wm, bm, gamma, beta):
    def body(p_ref, d0_ref, cnt_ref, pr_ref, wm_ref, bm_ref, g_ref, be_ref,
             o_ref, spad):
        s = _merge_windows(p_ref, d0_ref, spad)
        mean = s / jnp.maximum(cnt_ref[...], 1.0)
        x6 = jnp.tanh(mean)
        nrm = jnp.sqrt(jnp.sum(x6 * x6, axis=1, keepdims=True))
        xn = x6 / (nrm + 1e-12)
        cu = xn[805:_N, :]
        cib = xn[:805, :]
        pr = pr_ref[...]
        es = lax.dot_general(pr, cu, (((1,), (0,)), ((), ())),
                             precision=_DEF, preferred_element_type=_F32)
        psum = jnp.sum(pr, axis=1, keepdims=True)
        e = es / psum
        e = lax.dot_general(e, wm_ref[...], (((1,), (1,)), ((), ())),
                            precision=_DEF, preferred_element_type=_F32)
        e = e + bm_ref[...]
        mu = jnp.mean(e, axis=0, keepdims=True)
        var = jnp.mean((e - mu) * (e - mu), axis=0, keepdims=True)
        e = (e - mu) / jnp.sqrt(var + 1e-5) * g_ref[...] + be_ref[...]
        e = jnp.maximum(e, 0.0)
        o_ref[...] = lax.dot_general(e, cib, (((1,), (1,)), ((), ())),
                                     precision=_DEF,
                                     preferred_element_type=_F32)
    return pl.pallas_call(
        body,
        out_shape=jax.ShapeDtypeStruct((1024, 805), _F32),
        scratch_shapes=[pltpu.VMEM((_N + _DW, 64), _F32)],
    )(p2, d0s, cnt_col, presc, Wm, bm, gamma, beta)


def kernel(edge_index_SH, prescription, total_g, sub_g1, sub_g2, emb, W1, b1,
           W2, b2, Wm, bm, gamma, beta):
    src = total_g[0]
    dst = total_g[1]
    perm = jnp.argsort(dst, stable=True)
    ssrc = jnp.pad(src[perm], (0, _EPAD - _E))
    sdst = jnp.pad(dst[perm], (0, _EPAD - _E))
    d0s = ((sdst[jnp.asarray(_BOUNDS, jnp.int32)] // 8) * 8).reshape(32, 1)
    zeros_hbm = jnp.zeros((_ACC,), _F32)

    Pc = _sc_cnt_hist(jnp.pad(dst, (0, _HEPAD - _E)), zeros_hbm)
    cnt_col = (Pc[:_CB] + Pc[_CB:])[:_N].reshape(_N, 1)

    m1 = _tc_lin(emb, W1, b1.reshape(1, 64))
    p1 = _sc_segscan(ssrc, sdst, m1.reshape(-1), zeros_hbm).reshape(32, _DW, 64)
    m2 = _tc_mid(p1, d0s, cnt_col, W2, b2.reshape(1, 64))
    p2 = _sc_segscan(ssrc, sdst, m2.reshape(-1), zeros_hbm).reshape(32, _DW, 64)
    return _tc_tail(p2, d0s, cnt_col, prescription, Wm, bm.reshape(1, 64),
                    gamma.reshape(1, 64), beta.reshape(1, 64))
